# baseline (speedup 1.0000x reference)
import functools

import jax
import jax.numpy as jnp
from jax import lax
from jax.experimental import pallas as pl
from jax.experimental.pallas import tpu as pltpu
from jax.experimental.pallas import tpu_sc as plsc

NUM_SEGMENTS = 256
E = 320000
D = 128

NC = 2
NS = 16
NW = NC * NS
ROWS_PER_TILE = E // NW
BLK = 128
NFULL = ROWS_PER_TILE // BLK
TAIL = ROWS_PER_TILE - NFULL * BLK
IDROWS = NFULL + 2
SEGS_PER_TILE = NUM_SEGMENTS // NS
RUNROLL = 4
NGRP2 = NFULL // 6

_mesh = plsc.VectorSubcoreMesh(core_axis_name="c", subcore_axis_name="s")


def _seg_sum_body(feat, ids2, out, fbuf, sbuf, tbuf, ibuf, iibuf, pacc, acc,
                  gs0, gs1, gb0, gb1, sb0, sb1, semi):
    c = lax.axis_index("c")
    s = lax.axis_index("s")
    gsems = (gs0, gs1)
    gbsems = (gb0, gb1)
    sbsems = (sb0, sb1)
    w = s * NC + c
    base = w * ROWS_PER_TILE

    pltpu.async_copy(ids2.at[pl.ds(w * IDROWS, IDROWS)], ibuf, semi)
    pltpu.async_copy(
        feat.at[pl.ds(base + NFULL * BLK, TAIL), :],
        tbuf.at[pl.ds(0, TAIL)], semi)

    zero16 = jnp.zeros((16,), jnp.float32)
    for r in range(TAIL, BLK):
        for j in range(D // 16):
            tbuf[r, pl.ds(j * 16, 16)] = zero16

    iota16 = lax.iota(jnp.int32, 16)
    for k in range(NUM_SEGMENTS // BLK):
        for j in range(BLK // 16):
            iibuf[k, pl.ds(j * 16, 16)] = iota16 + (k * BLK + j * 16)

    def zero_pacc(r, carry):
        for j in range(D // 16):
            pacc[r, pl.ds(j * 16, 16)] = zero16
        return carry

    lax.fori_loop(0, NUM_SEGMENTS, zero_pacc, None)
    seg0 = s * SEGS_PER_TILE
    pltpu.sync_copy(
        pacc.at[pl.ds(seg0, SEGS_PER_TILE)],
        acc.at[pl.ds(seg0, SEGS_PER_TILE)])
    plsc.subcore_barrier()

    def start_gather(i, buf, sem):
        pltpu.async_copy(feat.at[pl.ds(base + i * BLK, BLK), :], buf, sem)

    def wait_gather(buf, sem):
        pltpu.make_async_copy(feat.at[pl.ds(0, BLK), :], buf, sem).wait()

    start_gather(0, fbuf.at[0], gs0)
    start_gather(1, fbuf.at[1], gs1)
    start_gather(2, sbuf.at[0], gb0)

    pltpu.make_async_copy(ids2.at[pl.ds(0, IDROWS)], ibuf, semi).wait()
    pltpu.make_async_copy(
        feat.at[pl.ds(0, TAIL), :], tbuf.at[pl.ds(0, TAIL)], semi).wait()

    def accum_rows(fb, lo, hi, seg):
        def row_body(r, regs):
            return tuple(
                regs[j] + fb[r, pl.ds(j * 16, 16)] for j in range(D // 16))

        regs = lax.fori_loop(
            lo, hi, row_body,
            tuple(jnp.zeros((16,), jnp.float32) for _ in range(D // 16)))
        for j in range(D // 16):
            pacc[seg, pl.ds(j * 16, 16)] = (
                pacc[seg, pl.ds(j * 16, 16)] + regs[j])

    def process_dense(i, b):
        fb = fbuf.at[b]
        m = ibuf[i, pl.ds(0, 16)][0]
        mx = ibuf[i, pl.ds(BLK - 16, 16)][15]

        @pl.when(m == mx)
        def _uniform():
            def row_body(it, regs):
                new = regs
                for u in range(RUNROLL):
                    r = it * RUNROLL + u
                    new = tuple(
                        new[j] + fb[r, pl.ds(j * 16, 16)]
                        for j in range(D // 16))
                return new

            regs = lax.fori_loop(
                0, BLK // RUNROLL, row_body,
                tuple(jnp.zeros((16,), jnp.float32)
                      for _ in range(D // 16)))
            for j in range(D // 16):
                pacc[m, pl.ds(j * 16, 16)] = (
                    pacc[m, pl.ds(j * 16, 16)] + regs[j])

        @pl.when(m != mx)
        def _boundary():
            pltpu.sync_copy(fb, acc.at[ibuf.at[i]], add=True)

    def group(i, ks):
        wait_gather(fbuf.at[0], gs0)
        process_dense(i, 0)

        @pl.when(i + 3 < NFULL)
        def _pf0():
            start_gather(i + 3, fbuf.at[0], gs0)

        wait_gather(fbuf.at[1], gs1)
        process_dense(i + 1, 1)

        @pl.when(i + 4 < NFULL)
        def _pf1():
            start_gather(i + 4, fbuf.at[1], gs1)

        j = i + 2
        wait_gather(sbuf.at[ks], gbsems[ks])
        pltpu.async_copy(
            sbuf.at[ks], acc.at[ibuf.at[j]], sbsems[ks], add=True)

        @pl.when(j >= 5)
        def _drain_prev():
            pltpu.make_async_copy(
                sbuf.at[1 - ks], acc.at[iibuf.at[0]], sbsems[1 - ks]).wait()

        @pl.when(j < NFULL - 3)
        def _pf2():
            start_gather(j + 3, sbuf.at[1 - ks], gbsems[1 - ks])

    def loop_body(iv2, carry):
        group(6 * iv2, 0)
        group(6 * iv2 + 3, 1)
        return carry

    lax.fori_loop(0, NGRP2, loop_body, None)

    pltpu.make_async_copy(sbuf.at[1], acc.at[iibuf.at[0]], sb1).wait()

    pltpu.sync_copy(tbuf, acc.at[ibuf.at[NFULL]], add=True)

    for k in range(NUM_SEGMENTS // BLK):
        pltpu.sync_copy(
            pacc.at[pl.ds(k * BLK, BLK)], acc.at[iibuf.at[k]], add=True)

    plsc.subcore_barrier()
    pltpu.sync_copy(
        acc.at[pl.ds(seg0, SEGS_PER_TILE)],
        out.at[c, pl.ds(seg0, SEGS_PER_TILE), :])


_seg_sum = pl.kernel(
    _seg_sum_body,
    out_type=jax.ShapeDtypeStruct((NC, NUM_SEGMENTS, D), jnp.float32),
    mesh=_mesh,
    scratch_types=[
        pltpu.VMEM((2, BLK, D), jnp.float32),
        pltpu.VMEM((2, BLK, D), jnp.float32),
        pltpu.VMEM((BLK, D), jnp.float32),
        pltpu.VMEM((IDROWS, BLK), jnp.int32),
        pltpu.VMEM((NUM_SEGMENTS // BLK, BLK), jnp.int32),
        pltpu.VMEM((NUM_SEGMENTS, D), jnp.float32),
        pltpu.VMEM_SHARED((NUM_SEGMENTS, D), jnp.float32),
        pltpu.SemaphoreType.DMA,
        pltpu.SemaphoreType.DMA,
        pltpu.SemaphoreType.DMA,
        pltpu.SemaphoreType.DMA,
        pltpu.SemaphoreType.DMA,
        pltpu.SemaphoreType.DMA,
        pltpu.SemaphoreType.DMA,
    ],
)


def _combine_body(p_ref, o_ref):
    o_ref[...] = p_ref[0] + p_ref[1]


_combine = pl.pallas_call(
    _combine_body,
    out_shape=jax.ShapeDtypeStruct((NUM_SEGMENTS, D), jnp.float32),
)


def kernel(feat, segment_ids):
    ids2 = jnp.pad(
        segment_ids.reshape(NW, ROWS_PER_TILE),
        ((0, 0), (0, IDROWS * BLK - ROWS_PER_TILE)),
    ).reshape(NW * IDROWS, BLK)
    partials = _seg_sum(feat, ids2)
    return _combine(partials)
